# baseline (speedup 1.0000x reference)
import functools

import jax
import jax.numpy as jnp
from jax import lax
from jax.experimental import pallas as pl
from jax.experimental.pallas import tpu as pltpu
from jax.experimental.pallas import tpu_sc as plsc

N = 10000
E = 160000
D = 128
NS = 16
EPT = E // NS
K = 80
NCHUNKS = EPT // K
SUPS = (50, 50, 25)
RPT = 624
TAIL = N - NS * RPT


def _copy_stripe(s, src, dst):
    r0 = s * RPT
    pltpu.sync_copy(src.at[pl.ds(r0, RPT)], dst.at[pl.ds(r0, RPT)])

    @pl.when(s == NS - 1)
    def _():
        pltpu.sync_copy(src.at[pl.ds(NS * RPT, TAIL)],
                        dst.at[pl.ds(NS * RPT, TAIL)])


def _super_pipeline(nch, attr0, x_hbm, attr_hbm, sidx, didx, rows, attrb,
                    semg, sema, semsc, acc):
    def issue_ga(j, b):
        pltpu.async_copy(x_hbm.at[sidx.at[pl.ds(j * K, K)]], rows[b],
                         semg[b])
        pltpu.async_copy(attr_hbm.at[pl.ds(attr0 + j * K, K)], attrb[b],
                         sema[b])

    def wait_ga(b):
        pltpu.make_async_copy(
            x_hbm.at[sidx.at[pl.ds(0, K)]], rows[b], semg[b]).wait()
        pltpu.make_async_copy(
            attr_hbm.at[pl.ds(0, K)], attrb[b], sema[b]).wait()

    def combine(b):
        def row_body(i, carry):
            for l in range(D // 16):
                plsc.addupdate(rows[b].at[i, pl.ds(16 * l, 16)],
                               attrb[b][i, pl.ds(16 * l, 16)])
            return carry

        lax.fori_loop(0, K, row_body, 0)

    def issue_sc(j, b):
        pltpu.async_copy(rows[b], acc.at[didx.at[pl.ds(j * K, K)]],
                         semsc[b], add=True)

    def wait_sc(b):
        pltpu.make_async_copy(
            rows[b], acc.at[didx.at[pl.ds(0, K)]], semsc[b]).wait()

    issue_ga(0, 0)
    wait_ga(0)
    issue_ga(1, 1)
    combine(0)
    issue_sc(0, 0)

    def body(i, carry):
        j = 2 * i + 1
        wait_ga(1)
        wait_sc(0)
        issue_ga(j + 1, 0)
        combine(1)
        issue_sc(j, 1)
        wait_ga(0)
        wait_sc(1)
        issue_ga(j + 2, 1)
        combine(0)
        issue_sc(j + 1, 0)
        return carry

    npairs = (nch - 2) // 2
    lax.fori_loop(0, npairs, body, 0)

    if nch % 2 == 0:
        wait_ga(1)
        combine(1)
        issue_sc(nch - 1, 1)
        wait_sc(0)
        wait_sc(1)
    else:
        j = nch - 2
        wait_ga(1)
        wait_sc(0)
        issue_ga(j + 1, 0)
        combine(1)
        issue_sc(j, 1)
        wait_ga(0)
        wait_sc(1)
        combine(0)
        issue_sc(j + 1, 0)
        wait_sc(0)


def _process_level(s, x_hbm, usrc, udst, dsrc, ddst, up_attr, dn_attr,
                   out_hbm, sidx, didx, rows, attrb, semg, sema, semsc, acc):
    _copy_stripe(s, x_hbm, acc)
    plsc.subcore_barrier()

    base = s * EPT
    for src_hbm, dst_hbm, attr_hbm in ((usrc, udst, up_attr),
                                       (dsrc, ddst, dn_attr)):
        sup0 = 0
        for nch in SUPS:
            ne = nch * K
            eoff = base + sup0 * K
            pltpu.sync_copy(src_hbm.at[pl.ds(eoff, ne)],
                            sidx.at[pl.ds(0, ne)])
            pltpu.sync_copy(dst_hbm.at[pl.ds(eoff, ne)],
                            didx.at[pl.ds(0, ne)])
            _super_pipeline(nch, eoff, x_hbm, attr_hbm, sidx, didx,
                            rows, attrb, semg, sema, semsc, acc)
            sup0 += nch

    plsc.subcore_barrier()
    _copy_stripe(s, acc, out_hbm)


def _sc_body(x0, us0, ud0, ds0, dd0, ua0, da0,
             x1, us1, ud1, ds1, dd1, ua1, da1,
             out0, out1,
             sidx, didx, rows0, rows1, attr0, attr1,
             semg0, semg1, sema0, sema1, semsc0, semsc1, acc):
    c = lax.axis_index("c")
    s = lax.axis_index("s")
    rows = (rows0, rows1)
    attrb = (attr0, attr1)
    semg = (semg0, semg1)
    sema = (sema0, sema1)
    semsc = (semsc0, semsc1)

    @pl.when(c == 0)
    def _():
        _process_level(s, x0, us0, ud0, ds0, dd0, ua0, da0, out0,
                       sidx, didx, rows, attrb, semg, sema, semsc, acc)

    @pl.when(c == 1)
    def _():
        _process_level(s, x1, us1, ud1, ds1, dd1, ua1, da1, out1,
                       sidx, didx, rows, attrb, semg, sema, semsc, acc)


_sc_kernel = functools.partial(
    pl.kernel,
    out_type=(jax.ShapeDtypeStruct((N, D), jnp.float32),
              jax.ShapeDtypeStruct((N, D), jnp.float32)),
    mesh=plsc.VectorSubcoreMesh(core_axis_name="c", subcore_axis_name="s"),
    scratch_types=[
        pltpu.VMEM((SUPS[0] * K,), jnp.int32),
        pltpu.VMEM((SUPS[0] * K,), jnp.int32),
        pltpu.VMEM((K, D), jnp.float32),
        pltpu.VMEM((K, D), jnp.float32),
        pltpu.VMEM((K, D), jnp.float32),
        pltpu.VMEM((K, D), jnp.float32),
        pltpu.SemaphoreType.DMA,
        pltpu.SemaphoreType.DMA,
        pltpu.SemaphoreType.DMA,
        pltpu.SemaphoreType.DMA,
        pltpu.SemaphoreType.DMA,
        pltpu.SemaphoreType.DMA,
        pltpu.VMEM_SHARED((N, D), jnp.float32),
    ],
)(_sc_body)


def kernel(x0, up_index0, down_index0, up_attr0, down_attr0,
           x1, up_index1, down_index1, up_attr1, down_attr1):
    return _sc_kernel(
        x0, up_index0[0], up_index0[1], down_index0[0], down_index0[1],
        up_attr0, down_attr0,
        x1, up_index1[0], up_index1[1], down_index1[0], down_index1[1],
        up_attr1, down_attr1,
    )
